# baseline (speedup 1.0000x reference)
import functools
import math

import jax
import jax.numpy as jnp
from jax import lax
from jax.experimental import pallas as pl
from jax.experimental.pallas import tpu as pltpu
from jax.experimental.pallas import tpu_sc as plsc

D_MODEL = 64
EMB_SCALE = math.sqrt(D_MODEL)
VOCAB = 1_000_000
NUM_WORKERS = 32
CHUNK = 128
LANES = 16
NBUF = 2
ROWS_PER_ITER = 4

VB = 2048


def _pack_table_body(in_ref, out_ref):
    out_ref[:, :D_MODEL] = in_ref[...].T


def _pack_table(table_t):
    n_vb = (VOCAB + VB - 1) // VB
    return pl.pallas_call(
        _pack_table_body,
        grid=(n_vb,),
        in_specs=[pl.BlockSpec((D_MODEL, VB), lambda i: (0, i))],
        out_specs=pl.BlockSpec((VB, 2 * D_MODEL), lambda i: (i, 0)),
        out_shape=jax.ShapeDtypeStruct((VOCAB, 2 * D_MODEL), jnp.float32),
    )(table_t)


BB = 1024


def _unpack_out_body(in_ref, out_ref):
    out_ref[0] = in_ref[0].T


def _unpack_out(flat3):
    n_pos, n_b, _ = flat3.shape
    return pl.pallas_call(
        _unpack_out_body,
        grid=(n_pos, n_b // BB),
        in_specs=[pl.BlockSpec((1, BB, D_MODEL), lambda j, b: (j, b, 0))],
        out_specs=pl.BlockSpec((1, D_MODEL, BB), lambda j, b: (j, 0, b))
        ,
        out_shape=jax.ShapeDtypeStruct((n_pos, D_MODEL, n_b), jnp.float32),
    )(flat3)


def _emb_body(table_hbm, idx_hbm, out_hbm, idx_v,
              gb0, gb1, ob0, ob1,
              gs0, gs1, os0, os1,
              *, n_chunks, b_per_w):
    gbufs = (gb0, gb1)
    obufs = (ob0, ob1)
    gsems = (gs0, gs1)
    osems = (os0, os1)

    wid = lax.axis_index("s") * 2 + lax.axis_index("c")
    base = wid * b_per_w
    pltpu.sync_copy(idx_hbm.at[wid], idx_v)

    for b in range(NBUF):
        pltpu.make_async_copy(
            table_hbm.at[idx_v.at[b]], gbufs[b], gsems[b]).start()

    def outer(gg, carry):
        for b in range(NBUF):
            g = gg + b
            pltpu.make_async_copy(
                table_hbm.at[idx_v.at[g]], gbufs[b], gsems[b]).wait()

            @pl.when(gg > 0)
            def _drain():
                pltpu.make_async_copy(
                    obufs[b], out_hbm.at[pl.ds(base, CHUNK)], osems[b]).wait()

            def row_body(i, c2):
                for r in range(ROWS_PER_ITER):
                    row = i * ROWS_PER_ITER + r
                    for j in range(D_MODEL // LANES):
                        sl = pl.ds(j * LANES, LANES)
                        obufs[b][row, sl] = gbufs[b][row, sl] * EMB_SCALE
                return c2

            lax.fori_loop(0, CHUNK // ROWS_PER_ITER, row_body, 0)

            pltpu.make_async_copy(
                obufs[b], out_hbm.at[pl.ds(base + g * CHUNK, CHUNK)],
                osems[b]).start()

            @pl.when(gg < n_chunks - NBUF)
            def _prefetch():
                pltpu.make_async_copy(
                    table_hbm.at[idx_v.at[g + NBUF]], gbufs[b],
                    gsems[b]).start()

        return carry

    lax.fori_loop(0, n_chunks // NBUF, lambda t, c: outer(t * NBUF, c), 0)

    for b in range(NBUF):
        pltpu.make_async_copy(
            obufs[b], out_hbm.at[pl.ds(base, CHUNK)], osems[b]).wait()


def kernel(data, embedding_weight):
    n_b, n_pos = data.shape
    batch = data.size
    b_per_w = batch // NUM_WORKERS
    n_chunks = b_per_w // CHUNK

    idx = data.T.reshape(NUM_WORKERS, n_chunks, CHUNK).astype(jnp.int32)

    tab_lin = _pack_table(embedding_weight.T)

    mesh = plsc.VectorSubcoreMesh(core_axis_name="c", subcore_axis_name="s")
    gbuf = lambda: pltpu.VMEM((CHUNK, 2 * D_MODEL), jnp.float32)
    obuf = lambda: pltpu.VMEM((CHUNK, D_MODEL), jnp.float32)
    emb = functools.partial(
        pl.kernel,
        mesh=mesh,
        compiler_params=pltpu.CompilerParams(use_tc_tiling_on_sc=True),
        out_type=jax.ShapeDtypeStruct((batch, D_MODEL), jnp.float32),
        scratch_types=[pltpu.VMEM((n_chunks, CHUNK), jnp.int32)]
        + [gbuf() for _ in range(NBUF)]
        + [obuf() for _ in range(NBUF)]
        + [pltpu.SemaphoreType.DMA for _ in range(2 * NBUF)],
    )(functools.partial(_emb_body, n_chunks=n_chunks, b_per_w=b_per_w))

    flat = emb(tab_lin, idx)

    out_slabs = _unpack_out(flat.reshape(n_pos, n_b, D_MODEL))
    return out_slabs.transpose(2, 0, 1)


# speedup vs baseline: 1.4104x; 1.4104x over previous
import functools
import math

import jax
import jax.numpy as jnp
from jax import lax
from jax.experimental import pallas as pl
from jax.experimental.pallas import tpu as pltpu
from jax.experimental.pallas import tpu_sc as plsc

D_MODEL = 64
EMB_SCALE = math.sqrt(D_MODEL)
VOCAB = 1_000_000
NUM_WORKERS = 32
CHUNK = 128
LANES = 16
NBUF = 2
ROWS_PER_ITER = 4

VB = 8192


def _pack_table_body(in_ref, out_ref):
    out_ref[:, :D_MODEL] = in_ref[...].T


def _pack_table(table_t):
    n_vb = (VOCAB + VB - 1) // VB
    return pl.pallas_call(
        _pack_table_body,
        grid=(n_vb,),
        in_specs=[pl.BlockSpec((D_MODEL, VB), lambda i: (0, i))],
        out_specs=pl.BlockSpec((VB, 2 * D_MODEL), lambda i: (i, 0)),
        out_shape=jax.ShapeDtypeStruct((VOCAB, 2 * D_MODEL), jnp.float32),
    )(table_t)


BB = 2048


def _unpack_out_body(in_ref, out_ref):
    out_ref[0] = in_ref[0].T


def _unpack_out(flat3):
    n_pos, n_b, _ = flat3.shape
    return pl.pallas_call(
        _unpack_out_body,
        grid=(n_pos, n_b // BB),
        in_specs=[pl.BlockSpec((1, BB, D_MODEL), lambda j, b: (j, b, 0))],
        out_specs=pl.BlockSpec((1, D_MODEL, BB), lambda j, b: (j, 0, b))
        ,
        out_shape=jax.ShapeDtypeStruct((n_pos, D_MODEL, n_b), jnp.float32),
    )(flat3)


def _emb_body(table_hbm, idx_hbm, out_hbm, idx_v,
              gb0, gb1, ob0, ob1,
              gs0, gs1, os0, os1,
              *, n_chunks, b_per_w):
    gbufs = (gb0, gb1)
    obufs = (ob0, ob1)
    gsems = (gs0, gs1)
    osems = (os0, os1)

    wid = lax.axis_index("s") * 2 + lax.axis_index("c")
    base = wid * b_per_w
    pltpu.sync_copy(idx_hbm.at[wid], idx_v)

    for b in range(NBUF):
        pltpu.make_async_copy(
            table_hbm.at[idx_v.at[b]], gbufs[b], gsems[b]).start()

    def outer(gg, carry):
        for b in range(NBUF):
            g = gg + b
            pltpu.make_async_copy(
                table_hbm.at[idx_v.at[g]], gbufs[b], gsems[b]).wait()

            @pl.when(gg > 0)
            def _drain():
                pltpu.make_async_copy(
                    obufs[b], out_hbm.at[pl.ds(base, CHUNK)], osems[b]).wait()

            def row_body(i, c2):
                for r in range(ROWS_PER_ITER):
                    row = i * ROWS_PER_ITER + r
                    for j in range(D_MODEL // LANES):
                        sl = pl.ds(j * LANES, LANES)
                        obufs[b][row, sl] = gbufs[b][row, sl] * EMB_SCALE
                return c2

            lax.fori_loop(0, CHUNK // ROWS_PER_ITER, row_body, 0)

            pltpu.make_async_copy(
                obufs[b], out_hbm.at[pl.ds(base + g * CHUNK, CHUNK)],
                osems[b]).start()

            @pl.when(gg < n_chunks - NBUF)
            def _prefetch():
                pltpu.make_async_copy(
                    table_hbm.at[idx_v.at[g + NBUF]], gbufs[b],
                    gsems[b]).start()

        return carry

    lax.fori_loop(0, n_chunks // NBUF, lambda t, c: outer(t * NBUF, c), 0)

    for b in range(NBUF):
        pltpu.make_async_copy(
            obufs[b], out_hbm.at[pl.ds(base, CHUNK)], osems[b]).wait()


def kernel(data, embedding_weight):
    n_b, n_pos = data.shape
    batch = data.size
    b_per_w = batch // NUM_WORKERS
    n_chunks = b_per_w // CHUNK

    idx = data.T.reshape(NUM_WORKERS, n_chunks, CHUNK).astype(jnp.int32)

    tab_lin = _pack_table(embedding_weight.T)

    mesh = plsc.VectorSubcoreMesh(core_axis_name="c", subcore_axis_name="s")
    gbuf = lambda: pltpu.VMEM((CHUNK, 2 * D_MODEL), jnp.float32)
    obuf = lambda: pltpu.VMEM((CHUNK, D_MODEL), jnp.float32)
    emb = functools.partial(
        pl.kernel,
        mesh=mesh,
        compiler_params=pltpu.CompilerParams(use_tc_tiling_on_sc=True),
        out_type=jax.ShapeDtypeStruct((batch, D_MODEL), jnp.float32),
        scratch_types=[pltpu.VMEM((n_chunks, CHUNK), jnp.int32)]
        + [gbuf() for _ in range(NBUF)]
        + [obuf() for _ in range(NBUF)]
        + [pltpu.SemaphoreType.DMA for _ in range(2 * NBUF)],
    )(functools.partial(_emb_body, n_chunks=n_chunks, b_per_w=b_per_w))

    flat = emb(tab_lin, idx)

    out_slabs = _unpack_out(flat.reshape(n_pos, n_b, D_MODEL))
    return out_slabs.transpose(2, 0, 1)


# speedup vs baseline: 1.6156x; 1.1455x over previous
import functools
import math

import jax
import jax.numpy as jnp
from jax import lax
from jax.experimental import pallas as pl
from jax.experimental.pallas import tpu as pltpu
from jax.experimental.pallas import tpu_sc as plsc

D_MODEL = 64
EMB_SCALE = math.sqrt(D_MODEL)
VOCAB = 1_000_000
NUM_WORKERS = 32
CHUNK = 128
LANES = 16
NBUF = 2
ROWS_PER_ITER = 4

VB = 16384


def _pack_table_body(in_ref, out_ref):
    out_ref[:, :D_MODEL] = in_ref[...].T


def _pack_table(table_t):
    n_vb = (VOCAB + VB - 1) // VB
    return pl.pallas_call(
        _pack_table_body,
        grid=(n_vb,),
        in_specs=[pl.BlockSpec((D_MODEL, VB), lambda i: (0, i))],
        out_specs=pl.BlockSpec((VB, 2 * D_MODEL), lambda i: (i, 0)),
        out_shape=jax.ShapeDtypeStruct((VOCAB, 2 * D_MODEL), jnp.float32),
    )(table_t)


BB = 4096


def _unpack_out_body(in_ref, out_ref):
    out_ref[0] = in_ref[0].T


def _unpack_out(flat3):
    n_pos, n_b, _ = flat3.shape
    return pl.pallas_call(
        _unpack_out_body,
        grid=(n_pos, n_b // BB),
        in_specs=[pl.BlockSpec((1, BB, D_MODEL), lambda j, b: (j, b, 0))],
        out_specs=pl.BlockSpec((1, D_MODEL, BB), lambda j, b: (j, 0, b))
        ,
        out_shape=jax.ShapeDtypeStruct((n_pos, D_MODEL, n_b), jnp.float32),
    )(flat3)


def _emb_body(table_hbm, idx_hbm, out_hbm, idx_v,
              gb0, gb1, ob0, ob1,
              gs0, gs1, os0, os1,
              *, n_chunks, b_per_w):
    gbufs = (gb0, gb1)
    obufs = (ob0, ob1)
    gsems = (gs0, gs1)
    osems = (os0, os1)

    wid = lax.axis_index("s") * 2 + lax.axis_index("c")
    base = wid * b_per_w
    pltpu.sync_copy(idx_hbm.at[wid], idx_v)

    for b in range(NBUF):
        pltpu.make_async_copy(
            table_hbm.at[idx_v.at[b]], gbufs[b], gsems[b]).start()

    def outer(gg, carry):
        for b in range(NBUF):
            g = gg + b
            pltpu.make_async_copy(
                table_hbm.at[idx_v.at[g]], gbufs[b], gsems[b]).wait()

            @pl.when(gg > 0)
            def _drain():
                pltpu.make_async_copy(
                    obufs[b], out_hbm.at[pl.ds(base, CHUNK)], osems[b]).wait()

            def row_body(i, c2):
                for r in range(ROWS_PER_ITER):
                    row = i * ROWS_PER_ITER + r
                    for j in range(D_MODEL // LANES):
                        sl = pl.ds(j * LANES, LANES)
                        obufs[b][row, sl] = gbufs[b][row, sl] * EMB_SCALE
                return c2

            lax.fori_loop(0, CHUNK // ROWS_PER_ITER, row_body, 0)

            pltpu.make_async_copy(
                obufs[b], out_hbm.at[pl.ds(base + g * CHUNK, CHUNK)],
                osems[b]).start()

            @pl.when(gg < n_chunks - NBUF)
            def _prefetch():
                pltpu.make_async_copy(
                    table_hbm.at[idx_v.at[g + NBUF]], gbufs[b],
                    gsems[b]).start()

        return carry

    lax.fori_loop(0, n_chunks // NBUF, lambda t, c: outer(t * NBUF, c), 0)

    for b in range(NBUF):
        pltpu.make_async_copy(
            obufs[b], out_hbm.at[pl.ds(base, CHUNK)], osems[b]).wait()


def kernel(data, embedding_weight):
    n_b, n_pos = data.shape
    batch = data.size
    b_per_w = batch // NUM_WORKERS
    n_chunks = b_per_w // CHUNK

    idx = data.T.reshape(NUM_WORKERS, n_chunks, CHUNK).astype(jnp.int32)

    tab_lin = _pack_table(embedding_weight.T)

    mesh = plsc.VectorSubcoreMesh(core_axis_name="c", subcore_axis_name="s")
    gbuf = lambda: pltpu.VMEM((CHUNK, 2 * D_MODEL), jnp.float32)
    obuf = lambda: pltpu.VMEM((CHUNK, D_MODEL), jnp.float32)
    emb = functools.partial(
        pl.kernel,
        mesh=mesh,
        compiler_params=pltpu.CompilerParams(use_tc_tiling_on_sc=True),
        out_type=jax.ShapeDtypeStruct((batch, D_MODEL), jnp.float32),
        scratch_types=[pltpu.VMEM((n_chunks, CHUNK), jnp.int32)]
        + [gbuf() for _ in range(NBUF)]
        + [obuf() for _ in range(NBUF)]
        + [pltpu.SemaphoreType.DMA for _ in range(2 * NBUF)],
    )(functools.partial(_emb_body, n_chunks=n_chunks, b_per_w=b_per_w))

    flat = emb(tab_lin, idx)

    out_slabs = _unpack_out(flat.reshape(n_pos, n_b, D_MODEL))
    return out_slabs.transpose(2, 0, 1)


# speedup vs baseline: 1.6300x; 1.0089x over previous
import functools
import math

import jax
import jax.numpy as jnp
from jax import lax
from jax.experimental import pallas as pl
from jax.experimental.pallas import tpu as pltpu
from jax.experimental.pallas import tpu_sc as plsc

D_MODEL = 64
EMB_SCALE = math.sqrt(D_MODEL)
VOCAB = 1_000_000
NUM_WORKERS = 32
CHUNK = 128
LANES = 16
NBUF = 2
ROWS_PER_ITER = 4

VB = 32768


def _pack_table_body(in_ref, out_ref):
    out_ref[:, :D_MODEL] = in_ref[...].T


def _pack_table(table_t):
    n_vb = (VOCAB + VB - 1) // VB
    return pl.pallas_call(
        _pack_table_body,
        grid=(n_vb,),
        in_specs=[pl.BlockSpec((D_MODEL, VB), lambda i: (0, i))],
        out_specs=pl.BlockSpec((VB, 2 * D_MODEL), lambda i: (i, 0)),
        out_shape=jax.ShapeDtypeStruct((VOCAB, 2 * D_MODEL), jnp.float32),
    )(table_t)


BB = 4096


def _unpack_out_body(in_ref, out_ref):
    out_ref[0] = in_ref[0].T


def _unpack_out(flat3):
    n_pos, n_b, _ = flat3.shape
    return pl.pallas_call(
        _unpack_out_body,
        grid=(n_pos, n_b // BB),
        in_specs=[pl.BlockSpec((1, BB, D_MODEL), lambda j, b: (j, b, 0))],
        out_specs=pl.BlockSpec((1, D_MODEL, BB), lambda j, b: (j, 0, b))
        ,
        out_shape=jax.ShapeDtypeStruct((n_pos, D_MODEL, n_b), jnp.float32),
    )(flat3)


def _emb_body(table_hbm, idx_hbm, out_hbm, idx_v,
              gb0, gb1, ob0, ob1,
              gs0, gs1, os0, os1,
              *, n_chunks, b_per_w):
    gbufs = (gb0, gb1)
    obufs = (ob0, ob1)
    gsems = (gs0, gs1)
    osems = (os0, os1)

    wid = lax.axis_index("s") * 2 + lax.axis_index("c")
    base = wid * b_per_w
    pltpu.sync_copy(idx_hbm.at[wid], idx_v)

    for b in range(NBUF):
        pltpu.make_async_copy(
            table_hbm.at[idx_v.at[b]], gbufs[b], gsems[b]).start()

    def outer(gg, carry):
        for b in range(NBUF):
            g = gg + b
            pltpu.make_async_copy(
                table_hbm.at[idx_v.at[g]], gbufs[b], gsems[b]).wait()

            @pl.when(gg > 0)
            def _drain():
                pltpu.make_async_copy(
                    obufs[b], out_hbm.at[pl.ds(base, CHUNK)], osems[b]).wait()

            def row_body(i, c2):
                for r in range(ROWS_PER_ITER):
                    row = i * ROWS_PER_ITER + r
                    for j in range(D_MODEL // LANES):
                        sl = pl.ds(j * LANES, LANES)
                        obufs[b][row, sl] = gbufs[b][row, sl] * EMB_SCALE
                return c2

            lax.fori_loop(0, CHUNK // ROWS_PER_ITER, row_body, 0)

            pltpu.make_async_copy(
                obufs[b], out_hbm.at[pl.ds(base + g * CHUNK, CHUNK)],
                osems[b]).start()

            @pl.when(gg < n_chunks - NBUF)
            def _prefetch():
                pltpu.make_async_copy(
                    table_hbm.at[idx_v.at[g + NBUF]], gbufs[b],
                    gsems[b]).start()

        return carry

    lax.fori_loop(0, n_chunks // NBUF, lambda t, c: outer(t * NBUF, c), 0)

    for b in range(NBUF):
        pltpu.make_async_copy(
            obufs[b], out_hbm.at[pl.ds(base, CHUNK)], osems[b]).wait()


def kernel(data, embedding_weight):
    n_b, n_pos = data.shape
    batch = data.size
    b_per_w = batch // NUM_WORKERS
    n_chunks = b_per_w // CHUNK

    idx = data.T.reshape(NUM_WORKERS, n_chunks, CHUNK).astype(jnp.int32)

    tab_lin = _pack_table(embedding_weight.T)

    mesh = plsc.VectorSubcoreMesh(core_axis_name="c", subcore_axis_name="s")
    gbuf = lambda: pltpu.VMEM((CHUNK, 2 * D_MODEL), jnp.float32)
    obuf = lambda: pltpu.VMEM((CHUNK, D_MODEL), jnp.float32)
    emb = functools.partial(
        pl.kernel,
        mesh=mesh,
        compiler_params=pltpu.CompilerParams(use_tc_tiling_on_sc=True),
        out_type=jax.ShapeDtypeStruct((batch, D_MODEL), jnp.float32),
        scratch_types=[pltpu.VMEM((n_chunks, CHUNK), jnp.int32)]
        + [gbuf() for _ in range(NBUF)]
        + [obuf() for _ in range(NBUF)]
        + [pltpu.SemaphoreType.DMA for _ in range(2 * NBUF)],
    )(functools.partial(_emb_body, n_chunks=n_chunks, b_per_w=b_per_w))

    flat = emb(tab_lin, idx)

    out_slabs = _unpack_out(flat.reshape(n_pos, n_b, D_MODEL))
    return out_slabs.transpose(2, 0, 1)
